# baseline (speedup 1.0000x reference)
import functools

import jax
import jax.numpy as jnp
from jax import lax
from jax.experimental import pallas as pl
from jax.experimental.pallas import tpu as pltpu
from jax.experimental.pallas import tpu_sc as plsc

_NUM_CORES = 2
_NUM_SUBCORES = 16
_NUM_WORKERS = _NUM_CORES * _NUM_SUBCORES
_CHUNK = 1280


def _gather_body(n_chunks, table_hbm, values_hbm, out_hbm,
                 idx_v, rows0, rows1, gsem, osem):
    wid = lax.axis_index("s") * _NUM_CORES + lax.axis_index("c")
    b_per_w = n_chunks * _CHUNK
    base = wid * b_per_w
    rows = (rows0, rows1)
    pltpu.sync_copy(values_hbm.at[pl.ds(base, b_per_w)], idx_v)

    def gather_start(j):
        idx_slice = idx_v.at[pl.ds(j * _CHUNK, _CHUNK)]
        return pltpu.async_copy(table_hbm.at[idx_slice], rows[j % 2], gsem)

    def out_start(j):
        return pltpu.async_copy(
            rows[j % 2], out_hbm.at[pl.ds(base + j * _CHUNK, _CHUNK)], osem)

    copies = {}
    copies[0] = gather_start(0)
    for j in range(n_chunks):
        copies[j].wait()
        if j >= 1:
            copies[(j - 1, "o")].wait()
        if j + 1 < n_chunks:
            copies[j + 1] = gather_start(j + 1)
        copies[(j, "o")] = out_start(j)
    copies[(n_chunks - 1, "o")].wait()


def kernel(table, values, lengths):
    total, dim = values.shape[0], table.shape[1]
    assert total % (_NUM_WORKERS * _CHUNK) == 0
    n_chunks = total // (_NUM_WORKERS * _CHUNK)
    mesh = plsc.VectorSubcoreMesh(core_axis_name="c", subcore_axis_name="s")
    run = pl.kernel(
        functools.partial(_gather_body, n_chunks),
        out_type=jax.ShapeDtypeStruct((total, dim), table.dtype),
        mesh=mesh,
        scratch_types=[
            pltpu.VMEM((n_chunks * _CHUNK,), jnp.int32),
            pltpu.VMEM((_CHUNK, dim), jnp.float32),
            pltpu.VMEM((_CHUNK, dim), jnp.float32),
            pltpu.SemaphoreType.DMA,
            pltpu.SemaphoreType.DMA,
        ],
        compiler_params=pltpu.CompilerParams(use_tc_tiling_on_sc=False),
    )
    emb = run(table, values)
    return (emb, lengths)
